# Initial kernel scaffold; baseline (speedup 1.0000x reference)
import jax
import jax.numpy as jnp
from jax.experimental import pallas as pl


def kernel(x, edge_index, W1, b1, W2, b2):
    raise NotImplementedError("write your pallas kernel here")


# speedup vs baseline: 10.2182x; 10.2182x over previous
import functools

import jax
import jax.numpy as jnp
from jax import lax
from jax.experimental import pallas as pl
from jax.experimental.pallas import tpu as pltpu
from jax.experimental.pallas import tpu_sc as plsc

N = 10000
E = 320000
IN_D = 128
HID = 256
OUT_D = 128

NC = 2
NS = 16
CH = 80
NRC = N // CH


def _mesh():
  return plsc.VectorSubcoreMesh(core_axis_name="c", subcore_axis_name="s",
                                num_cores=NC, num_subcores=NS)


def _row_chunks(tid, fn):
  for j in range(-(-NRC // NS)):
    idx = j * NS + tid

    @pl.when(idx < NRC)
    def _():
      fn(idx * CH)


def _fill(ref, n_rows, width, value):
  def body(i, _):
    for j in range(width // 16):
      ref[i, pl.ds(j * 16, 16)] = jnp.full((16,), value, jnp.float32)
    return 0
  lax.fori_loop(0, n_rows, body, 0)


def _deg_partials(dst):
  ept = E // (NC * NS)
  nchk = ept // CH

  @functools.partial(
      pl.kernel,
      out_type=jax.ShapeDtypeStruct((NC, N, 128), jnp.float32),
      mesh=_mesh(),
      scratch_types=[
          pltpu.VMEM((CH,), jnp.int32),
          pltpu.VMEM((CH, 128), jnp.float32),
          pltpu.VMEM((CH, 128), jnp.float32),
          pltpu.VMEM_SHARED((N, 128), jnp.float32),
      ],
  )
  def k(dst_hbm, out_hbm, idx_v, ones_v, zero_v, acc_sh):
    core = lax.axis_index("c")
    tid = lax.axis_index("s")
    _fill(ones_v, CH, 128, 1.0)
    _fill(zero_v, CH, 128, 0.0)
    _row_chunks(tid, lambda r0: pltpu.sync_copy(
        zero_v, acc_sh.at[pl.ds(r0, CH), :]))
    plsc.subcore_barrier()
    base = (core * NS + tid) * ept

    def chunk(kk, _):
      e0 = base + kk * CH
      pltpu.sync_copy(dst_hbm.at[pl.ds(e0, CH)], idx_v)
      pltpu.sync_copy(ones_v, acc_sh.at[idx_v], add=True)
      return 0
    lax.fori_loop(0, nchk, chunk, 0)
    plsc.subcore_barrier()
    _row_chunks(tid, lambda r0: pltpu.sync_copy(
        acc_sh.at[pl.ds(r0, CH), :], out_hbm.at[core, pl.ds(r0, CH), :]))

  return k(dst)


def _agg(y, src, dst, feature_split):
  ept = E // NS if feature_split else E // (NC * NS)
  nchk = ept // CH

  @functools.partial(
      pl.kernel,
      out_type=jax.ShapeDtypeStruct((NC, N, 128), jnp.float32),
      mesh=_mesh(),
      scratch_types=[
          pltpu.VMEM((CH,), jnp.int32),
          pltpu.VMEM((CH,), jnp.int32),
          pltpu.VMEM((CH, 128), jnp.float32),
          pltpu.VMEM((CH, 128), jnp.float32),
          pltpu.VMEM_SHARED((N, 128), jnp.float32),
          pltpu.SemaphoreType.DMA,
      ],
  )
  def k(y_hbm, src_hbm, dst_hbm, out_hbm, srcv, dstv, rows_v, zero_v, acc_sh,
        sem):
    core = lax.axis_index("c")
    tid = lax.axis_index("s")
    _fill(zero_v, CH, 128, 0.0)
    _row_chunks(tid, lambda r0: pltpu.sync_copy(
        zero_v, acc_sh.at[pl.ds(r0, CH), :]))
    plsc.subcore_barrier()
    if feature_split:
      base = tid * ept
    else:
      base = (core * NS + tid) * ept

    def chunk(kk, _):
      e0 = base + kk * CH
      pltpu.sync_copy(src_hbm.at[pl.ds(e0, CH)], srcv)
      if feature_split:
        off = core * N
        for j in range(CH // 16):
          srcv[pl.ds(j * 16, 16)] = srcv[pl.ds(j * 16, 16)] + off
      pltpu.sync_copy(dst_hbm.at[pl.ds(e0, CH)], dstv)
      pltpu.async_copy(y_hbm.at[srcv], rows_v, sem).wait()
      pltpu.sync_copy(rows_v, acc_sh.at[dstv], add=True)
      return 0
    lax.fori_loop(0, nchk, chunk, 0)
    plsc.subcore_barrier()
    _row_chunks(tid, lambda r0: pltpu.sync_copy(
        acc_sh.at[pl.ds(r0, CH), :], out_hbm.at[core, pl.ds(r0, CH), :]))

  return k(y, src, dst)


BLK = 400


def _tc_layer1(x, w1, degp):
  def body(x_ref, w_ref, dp_ref, y_ref, ds_ref):
    deg = dp_ref[0, :, 0:1] + dp_ref[1, :, 0:1] + 1.0
    ds = lax.rsqrt(deg)
    xw = jnp.dot(x_ref[...], w_ref[...], preferred_element_type=jnp.float32)
    y = xw * ds
    y_ref[0, :, :] = y[:, :128]
    y_ref[1, :, :] = y[:, 128:]
    ds_ref[...] = ds

  return pl.pallas_call(
      body,
      grid=(N // BLK,),
      in_specs=[
          pl.BlockSpec((BLK, IN_D), lambda i: (i, 0)),
          pl.BlockSpec((IN_D, HID), lambda i: (0, 0)),
          pl.BlockSpec((NC, BLK, 128), lambda i: (0, i, 0)),
      ],
      out_specs=[
          pl.BlockSpec((NC, BLK, 128), lambda i: (0, i, 0)),
          pl.BlockSpec((BLK, 1), lambda i: (i, 0)),
      ],
      out_shape=[
          jax.ShapeDtypeStruct((NC, N, 128), jnp.float32),
          jax.ShapeDtypeStruct((N, 1), jnp.float32),
      ],
  )(x, w1, degp)


def _tc_layer2(ds, y1pair, acc1, b1r, w2):
  def body(ds_ref, y1_ref, a1_ref, b1_ref, w_ref, y2_ref):
    ds = ds_ref[...]
    pre = (a1_ref[...] + y1_ref[...]) * ds[None, :, :] + b1_ref[...]
    h = jnp.maximum(pre, 0.0)
    hf = jnp.concatenate([h[0], h[1]], axis=1)
    z = jnp.dot(hf, w_ref[...], preferred_element_type=jnp.float32)
    y2_ref[...] = z * ds

  return pl.pallas_call(
      body,
      grid=(N // BLK,),
      in_specs=[
          pl.BlockSpec((BLK, 1), lambda i: (i, 0)),
          pl.BlockSpec((NC, BLK, 128), lambda i: (0, i, 0)),
          pl.BlockSpec((NC, BLK, 128), lambda i: (0, i, 0)),
          pl.BlockSpec((NC, 1, 128), lambda i: (0, 0, 0)),
          pl.BlockSpec((HID, OUT_D), lambda i: (0, 0)),
      ],
      out_specs=pl.BlockSpec((BLK, OUT_D), lambda i: (i, 0)),
      out_shape=jax.ShapeDtypeStruct((N, OUT_D), jnp.float32),
  )(ds, y1pair, acc1, b1r, w2)


def _tc_layer3(ds, acc2, y2, b2r):
  def body(ds_ref, a2_ref, y2_ref, b2_ref, o_ref):
    o_ref[...] = ((a2_ref[0] + a2_ref[1] + y2_ref[...]) * ds_ref[...]
                  + b2_ref[...])

  return pl.pallas_call(
      body,
      grid=(N // BLK,),
      in_specs=[
          pl.BlockSpec((BLK, 1), lambda i: (i, 0)),
          pl.BlockSpec((NC, BLK, OUT_D), lambda i: (0, i, 0)),
          pl.BlockSpec((BLK, OUT_D), lambda i: (i, 0)),
          pl.BlockSpec((1, OUT_D), lambda i: (0, 0)),
      ],
      out_specs=pl.BlockSpec((BLK, OUT_D), lambda i: (i, 0)),
      out_shape=jax.ShapeDtypeStruct((N, OUT_D), jnp.float32),
  )(ds, acc2, y2, b2r)


def kernel(x, edge_index, W1, b1, W2, b2):
  src = edge_index[0].astype(jnp.int32)
  dst = edge_index[1].astype(jnp.int32)
  degp = _deg_partials(dst)
  y1pair, ds = _tc_layer1(x, W1, degp)
  acc1 = _agg(y1pair.reshape(NC * N, 128), src, dst, feature_split=True)
  y2 = _tc_layer2(ds, y1pair, acc1, b1.reshape(NC, 1, 128), W2)
  acc2 = _agg(y2, src, dst, feature_split=False)
  return _tc_layer3(ds, acc2, y2, b2.reshape(1, OUT_D))


# speedup vs baseline: 22.8327x; 2.2345x over previous
import functools

import jax
import jax.numpy as jnp
from jax import lax
from jax.experimental import pallas as pl
from jax.experimental.pallas import tpu as pltpu
from jax.experimental.pallas import tpu_sc as plsc

N = 10000
E = 320000
IN_D = 128
HID = 256
OUT_D = 128

NC = 2
NS = 16
CH = 80
NRC = N // CH


def _mesh():
  return plsc.VectorSubcoreMesh(core_axis_name="c", subcore_axis_name="s",
                                num_cores=NC, num_subcores=NS)


def _row_chunks(tid, fn):
  for j in range(-(-NRC // NS)):
    idx = j * NS + tid

    @pl.when(idx < NRC)
    def _():
      fn(idx * CH)


def _fill(ref, n_rows, width, value):
  def body(i, _):
    for j in range(width // 16):
      ref[i, pl.ds(j * 16, 16)] = jnp.full((16,), value, jnp.float32)
    return 0
  lax.fori_loop(0, n_rows, body, 0)


def _deg_partials(dst):
  ept = E // (NC * NS)
  nchk = ept // CH
  grp = 4

  @functools.partial(
      pl.kernel,
      out_type=jax.ShapeDtypeStruct((NC, N, 128), jnp.float32),
      mesh=_mesh(),
      scratch_types=[
          pltpu.VMEM((ept,), jnp.int32),
          pltpu.VMEM((CH, 128), jnp.float32),
          pltpu.VMEM((CH, 128), jnp.float32),
          pltpu.VMEM_SHARED((N, 128), jnp.float32),
          pltpu.SemaphoreType.DMA,
      ],
  )
  def k(dst_hbm, out_hbm, idx_v, ones_v, zero_v, acc_sh, sem):
    core = lax.axis_index("c")
    tid = lax.axis_index("s")

    def didx(c):
      return idx_v.at[pl.ds(c * CH, CH)]

    _fill(ones_v, CH, 128, 1.0)
    _fill(zero_v, CH, 128, 0.0)
    _row_chunks(tid, lambda r0: pltpu.sync_copy(
        zero_v, acc_sh.at[pl.ds(r0, CH), :]))
    pltpu.sync_copy(dst_hbm.at[pl.ds((core * NS + tid) * ept, ept)], idx_v)
    plsc.subcore_barrier()

    def group(g, _):
      for j in range(grp):
        pltpu.async_copy(ones_v, acc_sh.at[didx(g * grp + j)], sem, add=True)
      for j in range(grp):
        pltpu.make_async_copy(ones_v, acc_sh.at[didx(g * grp + j)],
                              sem).wait()
      return 0
    lax.fori_loop(0, nchk // grp, group, 0)
    for c in range(nchk - nchk % grp, nchk):
      pltpu.sync_copy(ones_v, acc_sh.at[didx(c)], add=True)
    plsc.subcore_barrier()
    _row_chunks(tid, lambda r0: pltpu.sync_copy(
        acc_sh.at[pl.ds(r0, CH), :], out_hbm.at[core, pl.ds(r0, CH), :]))

  return k(dst)


def _agg(y, src, dst, feature_split):
  ept = E // NS if feature_split else E // (NC * NS)
  halves = 2 if feature_split else 1
  hept = ept // halves
  nchk = hept // CH
  npairs = nchk // 2

  @functools.partial(
      pl.kernel,
      out_type=jax.ShapeDtypeStruct((NC, N, 128), jnp.float32),
      mesh=_mesh(),
      scratch_types=[
          pltpu.VMEM((hept,), jnp.int32),
          pltpu.VMEM((hept,), jnp.int32),
          pltpu.VMEM((CH, 128), jnp.float32),
          pltpu.VMEM((CH, 128), jnp.float32),
          pltpu.VMEM((CH, 128), jnp.float32),
          pltpu.VMEM_SHARED((N, 128), jnp.float32),
          pltpu.SemaphoreType.DMA,
          pltpu.SemaphoreType.DMA,
          pltpu.SemaphoreType.DMA,
          pltpu.SemaphoreType.DMA,
      ],
  )
  def k(y_hbm, src_hbm, dst_hbm, out_hbm, src_v, dst_v, rows_a, rows_b,
        zero_v, acc_sh, sga, sgb, ssa, ssb):
    core = lax.axis_index("c")
    tid = lax.axis_index("s")

    def sidx(c):
      return src_v.at[pl.ds(c * CH, CH)]

    def didx(c):
      return dst_v.at[pl.ds(c * CH, CH)]

    _fill(zero_v, CH, 128, 0.0)
    _row_chunks(tid, lambda r0: pltpu.sync_copy(
        zero_v, acc_sh.at[pl.ds(r0, CH), :]))
    base = tid * ept if feature_split else (core * NS + tid) * ept

    def g_start(c, rows, sem):
      pltpu.async_copy(y_hbm.at[sidx(c)], rows, sem)

    def g_wait(c, rows, sem):
      pltpu.make_async_copy(y_hbm.at[sidx(c)], rows, sem).wait()

    def s_start(c, rows, sem):
      pltpu.async_copy(rows, acc_sh.at[didx(c)], sem, add=True)

    def s_wait(c, rows, sem):
      pltpu.make_async_copy(rows, acc_sh.at[didx(c)], sem).wait()

    for h in range(halves):
      e0 = base + h * hept
      pltpu.sync_copy(src_hbm.at[pl.ds(e0, hept)], src_v)
      pltpu.sync_copy(dst_hbm.at[pl.ds(e0, hept)], dst_v)
      if feature_split:
        off = core * N

        def addoff(i, _):
          src_v[pl.ds(i * 16, 16)] = src_v[pl.ds(i * 16, 16)] + off
          return 0
        lax.fori_loop(0, hept // 16, addoff, 0)
      if h == 0:
        plsc.subcore_barrier()

      g_start(0, rows_a, sga)

      def pipe(i, _):
        c0 = 2 * i

        @pl.when(i > 0)
        def _():
          s_wait(c0 - 1, rows_b, ssb)
        g_start(c0 + 1, rows_b, sgb)
        g_wait(c0, rows_a, sga)
        s_start(c0, rows_a, ssa)
        s_wait(c0, rows_a, ssa)

        @pl.when(c0 + 2 < nchk)
        def _():
          g_start(c0 + 2, rows_a, sga)
        g_wait(c0 + 1, rows_b, sgb)
        s_start(c0 + 1, rows_b, ssb)
        return 0
      lax.fori_loop(0, npairs, pipe, 0)
      s_wait(2 * npairs - 1, rows_b, ssb)
      if nchk % 2:
        c = nchk - 1
        g_wait(c, rows_a, sga)
        pltpu.sync_copy(rows_a, acc_sh.at[didx(c)], add=True)
    plsc.subcore_barrier()
    _row_chunks(tid, lambda r0: pltpu.sync_copy(
        acc_sh.at[pl.ds(r0, CH), :], out_hbm.at[core, pl.ds(r0, CH), :]))

  return k(y, src, dst)


BLK = 400


def _tc_layer1(x, w1, degp):
  def body(x_ref, w_ref, dp_ref, y_ref, ds_ref):
    deg = dp_ref[0, :, 0:1] + dp_ref[1, :, 0:1] + 1.0
    ds = lax.rsqrt(deg)
    xw = jnp.dot(x_ref[...], w_ref[...], preferred_element_type=jnp.float32)
    y = xw * ds
    y_ref[0, :, :] = y[:, :128]
    y_ref[1, :, :] = y[:, 128:]
    ds_ref[...] = ds

  return pl.pallas_call(
      body,
      grid=(N // BLK,),
      in_specs=[
          pl.BlockSpec((BLK, IN_D), lambda i: (i, 0)),
          pl.BlockSpec((IN_D, HID), lambda i: (0, 0)),
          pl.BlockSpec((NC, BLK, 128), lambda i: (0, i, 0)),
      ],
      out_specs=[
          pl.BlockSpec((NC, BLK, 128), lambda i: (0, i, 0)),
          pl.BlockSpec((BLK, 1), lambda i: (i, 0)),
      ],
      out_shape=[
          jax.ShapeDtypeStruct((NC, N, 128), jnp.float32),
          jax.ShapeDtypeStruct((N, 1), jnp.float32),
      ],
  )(x, w1, degp)


def _tc_layer2(ds, y1pair, acc1, b1r, w2):
  def body(ds_ref, y1_ref, a1_ref, b1_ref, w_ref, y2_ref):
    ds = ds_ref[...]
    pre = (a1_ref[...] + y1_ref[...]) * ds[None, :, :] + b1_ref[...]
    h = jnp.maximum(pre, 0.0)
    hf = jnp.concatenate([h[0], h[1]], axis=1)
    z = jnp.dot(hf, w_ref[...], preferred_element_type=jnp.float32)
    y2_ref[...] = z * ds

  return pl.pallas_call(
      body,
      grid=(N // BLK,),
      in_specs=[
          pl.BlockSpec((BLK, 1), lambda i: (i, 0)),
          pl.BlockSpec((NC, BLK, 128), lambda i: (0, i, 0)),
          pl.BlockSpec((NC, BLK, 128), lambda i: (0, i, 0)),
          pl.BlockSpec((NC, 1, 128), lambda i: (0, 0, 0)),
          pl.BlockSpec((HID, OUT_D), lambda i: (0, 0)),
      ],
      out_specs=pl.BlockSpec((BLK, OUT_D), lambda i: (i, 0)),
      out_shape=jax.ShapeDtypeStruct((N, OUT_D), jnp.float32),
  )(ds, y1pair, acc1, b1r, w2)


def _tc_layer3(ds, acc2, y2, b2r):
  def body(ds_ref, a2_ref, y2_ref, b2_ref, o_ref):
    o_ref[...] = ((a2_ref[0] + a2_ref[1] + y2_ref[...]) * ds_ref[...]
                  + b2_ref[...])

  return pl.pallas_call(
      body,
      grid=(N // BLK,),
      in_specs=[
          pl.BlockSpec((BLK, 1), lambda i: (i, 0)),
          pl.BlockSpec((NC, BLK, OUT_D), lambda i: (0, i, 0)),
          pl.BlockSpec((BLK, OUT_D), lambda i: (i, 0)),
          pl.BlockSpec((1, OUT_D), lambda i: (0, 0)),
      ],
      out_specs=pl.BlockSpec((BLK, OUT_D), lambda i: (i, 0)),
      out_shape=jax.ShapeDtypeStruct((N, OUT_D), jnp.float32),
  )(ds, acc2, y2, b2r)


def kernel(x, edge_index, W1, b1, W2, b2):
  src = edge_index[0].astype(jnp.int32)
  dst = edge_index[1].astype(jnp.int32)
  degp = _deg_partials(dst)
  y1pair, ds = _tc_layer1(x, W1, degp)
  acc1 = _agg(y1pair.reshape(NC * N, 128), src, dst, feature_split=True)
  y2 = _tc_layer2(ds, y1pair, acc1, b1.reshape(NC, 1, 128), W2)
  acc2 = _agg(y2, src, dst, feature_split=False)
  return _tc_layer3(ds, acc2, y2, b2.reshape(1, OUT_D))


# speedup vs baseline: 25.7737x; 1.1288x over previous
import functools

import jax
import jax.numpy as jnp
from jax import lax
from jax.experimental import pallas as pl
from jax.experimental.pallas import tpu as pltpu
from jax.experimental.pallas import tpu_sc as plsc

N = 10000
E = 320000
IN_D = 128
HID = 256
OUT_D = 128

NC = 2
NS = 16
CH = 80
NRC = N // CH


def _mesh():
  return plsc.VectorSubcoreMesh(core_axis_name="c", subcore_axis_name="s",
                                num_cores=NC, num_subcores=NS)


def _row_chunks(tid, fn):
  for j in range(-(-NRC // NS)):
    idx = j * NS + tid

    @pl.when(idx < NRC)
    def _():
      fn(idx * CH)


def _fill(ref, n_rows, width, value):
  def body(i, _):
    for j in range(width // 16):
      ref[i, pl.ds(j * 16, 16)] = jnp.full((16,), value, jnp.float32)
    return 0
  lax.fori_loop(0, n_rows, body, 0)


def _deg_partials(dst):
  ept = E // (NC * NS)
  nchk = ept // CH
  grp = 8

  @functools.partial(
      pl.kernel,
      out_type=jax.ShapeDtypeStruct((NC, N, 128), jnp.float32),
      mesh=_mesh(),
      scratch_types=[
          pltpu.VMEM((ept,), jnp.int32),
          pltpu.VMEM((CH, 128), jnp.float32),
          pltpu.VMEM((CH, 128), jnp.float32),
          pltpu.VMEM_SHARED((N, 128), jnp.float32),
          pltpu.SemaphoreType.DMA,
      ],
  )
  def k(dst_hbm, out_hbm, idx_v, ones_v, zero_v, acc_sh, sem):
    core = lax.axis_index("c")
    tid = lax.axis_index("s")

    def didx(c):
      return idx_v.at[pl.ds(c * CH, CH)]

    _fill(ones_v, CH, 128, 1.0)
    _fill(zero_v, CH, 128, 0.0)
    _row_chunks(tid, lambda r0: pltpu.sync_copy(
        zero_v, acc_sh.at[pl.ds(r0, CH), :]))
    pltpu.sync_copy(dst_hbm.at[pl.ds((core * NS + tid) * ept, ept)], idx_v)
    plsc.subcore_barrier()

    def group(g, _):
      for j in range(grp):
        pltpu.async_copy(ones_v, acc_sh.at[didx(g * grp + j)], sem, add=True)
      for j in range(grp):
        pltpu.make_async_copy(ones_v, acc_sh.at[didx(g * grp + j)],
                              sem).wait()
      return 0
    lax.fori_loop(0, nchk // grp, group, 0)
    for c in range(nchk - nchk % grp, nchk):
      pltpu.async_copy(ones_v, acc_sh.at[didx(c)], sem, add=True)
    for c in range(nchk - nchk % grp, nchk):
      pltpu.make_async_copy(ones_v, acc_sh.at[didx(c)], sem).wait()
    plsc.subcore_barrier()
    _row_chunks(tid, lambda r0: pltpu.async_copy(
        acc_sh.at[pl.ds(r0, CH), :], out_hbm.at[core, pl.ds(r0, CH), :],
        sem))
    _row_chunks(tid, lambda r0: pltpu.make_async_copy(
        acc_sh.at[pl.ds(r0, CH), :], out_hbm.at[core, pl.ds(r0, CH), :],
        sem).wait())

  return k(dst)


def _agg(y, src, dst, feature_split):
  ept = E // NS if feature_split else E // (NC * NS)
  halves = 2 if feature_split else 1
  hept = ept // halves
  nchk = hept // CH
  npip = (nchk - 2) // 3

  @functools.partial(
      pl.kernel,
      out_type=jax.ShapeDtypeStruct((NC, N, 128), jnp.float32),
      mesh=_mesh(),
      scratch_types=[
          pltpu.VMEM((hept,), jnp.int32),
          pltpu.VMEM((hept,), jnp.int32),
          pltpu.VMEM((CH, 128), jnp.float32),
          pltpu.VMEM((CH, 128), jnp.float32),
          pltpu.VMEM((CH, 128), jnp.float32),
          pltpu.VMEM_SHARED((N, 128), jnp.float32),
          [pltpu.SemaphoreType.DMA] * 3,
          [pltpu.SemaphoreType.DMA] * 3,
          pltpu.SemaphoreType.DMA,
      ],
  )
  def k(y_hbm, src_hbm, dst_hbm, out_hbm, src_v, dst_v, b0, b1, b2,
        acc_sh, sg, ss, sw):
    core = lax.axis_index("c")
    tid = lax.axis_index("s")
    bufs = (b0, b1, b2)

    def sidx(c):
      return src_v.at[pl.ds(c * CH, CH)]

    def didx(c):
      return dst_v.at[pl.ds(c * CH, CH)]

    _fill(b0, CH, 128, 0.0)
    _row_chunks(tid, lambda r0: pltpu.async_copy(
        b0, acc_sh.at[pl.ds(r0, CH), :], sw))
    _row_chunks(tid, lambda r0: pltpu.make_async_copy(
        b0, acc_sh.at[pl.ds(r0, CH), :], sw).wait())
    base = tid * ept if feature_split else (core * NS + tid) * ept

    def g_start(c, j):
      pltpu.async_copy(y_hbm.at[sidx(c)], bufs[j], sg[j])

    def g_wait(c, j):
      pltpu.make_async_copy(y_hbm.at[sidx(c)], bufs[j], sg[j]).wait()

    def s_start(c, j):
      pltpu.async_copy(bufs[j], acc_sh.at[didx(c)], ss[j], add=True)

    def s_wait(c, j):
      pltpu.make_async_copy(bufs[j], acc_sh.at[didx(c)], ss[j]).wait()

    for h in range(halves):
      e0 = base + h * hept
      pltpu.sync_copy(src_hbm.at[pl.ds(e0, hept)], src_v)
      pltpu.sync_copy(dst_hbm.at[pl.ds(e0, hept)], dst_v)
      if feature_split:
        off = core * N

        def addoff(i, _):
          src_v[pl.ds(i * 16, 16)] = src_v[pl.ds(i * 16, 16)] + off
          return 0
        lax.fori_loop(0, hept // 16, addoff, 0)
      if h == 0:
        plsc.subcore_barrier()

      g_start(0, 0)
      g_start(1, 1)

      def pipe(i, _):
        for j in range(3):
          c = 3 * i + j
          g_wait(c, j)

          @pl.when(c > 0)
          def _():
            s_wait(c - 1, (j + 2) % 3)

          @pl.when(c + 2 < nchk)
          def _():
            g_start(c + 2, (j + 2) % 3)
          s_start(c, j)
        return 0
      lax.fori_loop(0, npip, pipe, 0)
      for c in range(3 * npip, nchk):
        j = c % 3
        g_wait(c, j)
        s_wait(c - 1, (j + 2) % 3)
        s_start(c, j)
      s_wait(nchk - 1, (nchk - 1) % 3)
    plsc.subcore_barrier()
    _row_chunks(tid, lambda r0: pltpu.async_copy(
        acc_sh.at[pl.ds(r0, CH), :], out_hbm.at[core, pl.ds(r0, CH), :], sw))
    _row_chunks(tid, lambda r0: pltpu.make_async_copy(
        acc_sh.at[pl.ds(r0, CH), :], out_hbm.at[core, pl.ds(r0, CH), :],
        sw).wait())

  return k(y, src, dst)


BLK = 400


def _tc_layer1(x, w1, degp):
  def body(x_ref, w_ref, dp_ref, y_ref, ds_ref):
    deg = dp_ref[0, :, 0:1] + dp_ref[1, :, 0:1] + 1.0
    ds = lax.rsqrt(deg)
    xw = jnp.dot(x_ref[...], w_ref[...], preferred_element_type=jnp.float32)
    y = xw * ds
    y_ref[0, :, :] = y[:, :128]
    y_ref[1, :, :] = y[:, 128:]
    ds_ref[...] = ds

  return pl.pallas_call(
      body,
      grid=(N // BLK,),
      in_specs=[
          pl.BlockSpec((BLK, IN_D), lambda i: (i, 0)),
          pl.BlockSpec((IN_D, HID), lambda i: (0, 0)),
          pl.BlockSpec((NC, BLK, 128), lambda i: (0, i, 0)),
      ],
      out_specs=[
          pl.BlockSpec((NC, BLK, 128), lambda i: (0, i, 0)),
          pl.BlockSpec((BLK, 1), lambda i: (i, 0)),
      ],
      out_shape=[
          jax.ShapeDtypeStruct((NC, N, 128), jnp.float32),
          jax.ShapeDtypeStruct((N, 1), jnp.float32),
      ],
  )(x, w1, degp)


def _tc_layer2(ds, y1pair, acc1, b1r, w2):
  def body(ds_ref, y1_ref, a1_ref, b1_ref, w_ref, y2_ref):
    ds = ds_ref[...]
    pre = (a1_ref[...] + y1_ref[...]) * ds[None, :, :] + b1_ref[...]
    h = jnp.maximum(pre, 0.0)
    hf = jnp.concatenate([h[0], h[1]], axis=1)
    z = jnp.dot(hf, w_ref[...], preferred_element_type=jnp.float32)
    y2_ref[...] = z * ds

  return pl.pallas_call(
      body,
      grid=(N // BLK,),
      in_specs=[
          pl.BlockSpec((BLK, 1), lambda i: (i, 0)),
          pl.BlockSpec((NC, BLK, 128), lambda i: (0, i, 0)),
          pl.BlockSpec((NC, BLK, 128), lambda i: (0, i, 0)),
          pl.BlockSpec((NC, 1, 128), lambda i: (0, 0, 0)),
          pl.BlockSpec((HID, OUT_D), lambda i: (0, 0)),
      ],
      out_specs=pl.BlockSpec((BLK, OUT_D), lambda i: (i, 0)),
      out_shape=jax.ShapeDtypeStruct((N, OUT_D), jnp.float32),
  )(ds, y1pair, acc1, b1r, w2)


def _tc_layer3(ds, acc2, y2, b2r):
  def body(ds_ref, a2_ref, y2_ref, b2_ref, o_ref):
    o_ref[...] = ((a2_ref[0] + a2_ref[1] + y2_ref[...]) * ds_ref[...]
                  + b2_ref[...])

  return pl.pallas_call(
      body,
      grid=(N // BLK,),
      in_specs=[
          pl.BlockSpec((BLK, 1), lambda i: (i, 0)),
          pl.BlockSpec((NC, BLK, OUT_D), lambda i: (0, i, 0)),
          pl.BlockSpec((BLK, OUT_D), lambda i: (i, 0)),
          pl.BlockSpec((1, OUT_D), lambda i: (0, 0)),
      ],
      out_specs=pl.BlockSpec((BLK, OUT_D), lambda i: (i, 0)),
      out_shape=jax.ShapeDtypeStruct((N, OUT_D), jnp.float32),
  )(ds, acc2, y2, b2r)


def kernel(x, edge_index, W1, b1, W2, b2):
  src = edge_index[0].astype(jnp.int32)
  dst = edge_index[1].astype(jnp.int32)
  degp = _deg_partials(dst)
  y1pair, ds = _tc_layer1(x, W1, degp)
  acc1 = _agg(y1pair.reshape(NC * N, 128), src, dst, feature_split=True)
  y2 = _tc_layer2(ds, y1pair, acc1, b1.reshape(NC, 1, 128), W2)
  acc2 = _agg(y2, src, dst, feature_split=False)
  return _tc_layer3(ds, acc2, y2, b2.reshape(1, OUT_D))


# speedup vs baseline: 26.1109x; 1.0131x over previous
import functools

import jax
import jax.numpy as jnp
from jax import lax
from jax.experimental import pallas as pl
from jax.experimental.pallas import tpu as pltpu
from jax.experimental.pallas import tpu_sc as plsc

N = 10000
E = 320000
IN_D = 128
HID = 256
OUT_D = 128

NC = 2
NS = 16
CH = 80
NRC = N // CH


def _mesh():
  return plsc.VectorSubcoreMesh(core_axis_name="c", subcore_axis_name="s",
                                num_cores=NC, num_subcores=NS)


def _row_chunks(tid, fn):
  for j in range(-(-NRC // NS)):
    idx = j * NS + tid

    @pl.when(idx < NRC)
    def _():
      fn(idx * CH)


def _fill(ref, n_rows, width, value):
  def body(i, _):
    for j in range(width // 16):
      ref[i, pl.ds(j * 16, 16)] = jnp.full((16,), value, jnp.float32)
    return 0
  lax.fori_loop(0, n_rows, body, 0)


def _deg_partials(dst):
  ept = E // (NC * NS)
  nchk = ept // CH
  grp = 8

  @functools.partial(
      pl.kernel,
      out_type=jax.ShapeDtypeStruct((NC, N, 128), jnp.float32),
      mesh=_mesh(),
      scratch_types=[
          pltpu.VMEM((ept,), jnp.int32),
          pltpu.VMEM((CH, 128), jnp.float32),
          pltpu.VMEM((CH, 128), jnp.float32),
          pltpu.VMEM_SHARED((N, 128), jnp.float32),
          pltpu.SemaphoreType.DMA,
          pltpu.SemaphoreType.DMA,
      ],
  )
  def k(dst_hbm, out_hbm, idx_v, ones_v, zero_v, acc_sh, sem, sidxl):
    core = lax.axis_index("c")
    tid = lax.axis_index("s")

    def didx(c):
      return idx_v.at[pl.ds(c * CH, CH)]

    idx_src = dst_hbm.at[pl.ds((core * NS + tid) * ept, ept)]
    pltpu.async_copy(idx_src, idx_v, sidxl)
    _fill(ones_v, CH, 128, 1.0)
    _fill(zero_v, CH, 128, 0.0)
    _row_chunks(tid, lambda r0: pltpu.async_copy(
        zero_v, acc_sh.at[pl.ds(r0, CH), :], sem))
    _row_chunks(tid, lambda r0: pltpu.make_async_copy(
        zero_v, acc_sh.at[pl.ds(r0, CH), :], sem).wait())
    pltpu.make_async_copy(idx_src, idx_v, sidxl).wait()
    plsc.subcore_barrier()

    def group(g, _):
      for j in range(grp):
        pltpu.async_copy(ones_v, acc_sh.at[didx(g * grp + j)], sem, add=True)
      for j in range(grp):
        pltpu.make_async_copy(ones_v, acc_sh.at[didx(g * grp + j)],
                              sem).wait()
      return 0
    lax.fori_loop(0, nchk // grp, group, 0)
    for c in range(nchk - nchk % grp, nchk):
      pltpu.async_copy(ones_v, acc_sh.at[didx(c)], sem, add=True)
    for c in range(nchk - nchk % grp, nchk):
      pltpu.make_async_copy(ones_v, acc_sh.at[didx(c)], sem).wait()
    plsc.subcore_barrier()
    _row_chunks(tid, lambda r0: pltpu.async_copy(
        acc_sh.at[pl.ds(r0, CH), :], out_hbm.at[core, pl.ds(r0, CH), :],
        sem))
    _row_chunks(tid, lambda r0: pltpu.make_async_copy(
        acc_sh.at[pl.ds(r0, CH), :], out_hbm.at[core, pl.ds(r0, CH), :],
        sem).wait())

  return k(dst)


def _agg(y, src, dst, feature_split):
  ept = E // NS if feature_split else E // (NC * NS)
  halves = 2 if feature_split else 1
  hept = ept // halves
  nchk = hept // CH
  npip = (nchk - 2) // 3

  @functools.partial(
      pl.kernel,
      out_type=jax.ShapeDtypeStruct((NC, N, 128), jnp.float32),
      mesh=_mesh(),
      scratch_types=[
          pltpu.VMEM((hept,), jnp.int32),
          pltpu.VMEM((hept,), jnp.int32),
          pltpu.VMEM((CH, 128), jnp.float32),
          pltpu.VMEM((CH, 128), jnp.float32),
          pltpu.VMEM((CH, 128), jnp.float32),
          pltpu.VMEM_SHARED((N, 128), jnp.float32),
          [pltpu.SemaphoreType.DMA] * 3,
          [pltpu.SemaphoreType.DMA] * 3,
          pltpu.SemaphoreType.DMA,
      ],
  )
  def k(y_hbm, src_hbm, dst_hbm, out_hbm, src_v, dst_v, b0, b1, b2,
        acc_sh, sg, ss, sw):
    core = lax.axis_index("c")
    tid = lax.axis_index("s")
    bufs = (b0, b1, b2)

    def sidx(c):
      return src_v.at[pl.ds(c * CH, CH)]

    def didx(c):
      return dst_v.at[pl.ds(c * CH, CH)]

    base = tid * ept if feature_split else (core * NS + tid) * ept
    pltpu.async_copy(src_hbm.at[pl.ds(base, hept)], src_v, sg[0])
    pltpu.async_copy(dst_hbm.at[pl.ds(base, hept)], dst_v, sg[1])
    _fill(b0, CH, 128, 0.0)
    _row_chunks(tid, lambda r0: pltpu.async_copy(
        b0, acc_sh.at[pl.ds(r0, CH), :], sw))
    _row_chunks(tid, lambda r0: pltpu.make_async_copy(
        b0, acc_sh.at[pl.ds(r0, CH), :], sw).wait())
    pltpu.make_async_copy(src_hbm.at[pl.ds(base, hept)], src_v, sg[0]).wait()
    pltpu.make_async_copy(dst_hbm.at[pl.ds(base, hept)], dst_v, sg[1]).wait()

    def g_start(c, j):
      pltpu.async_copy(y_hbm.at[sidx(c)], bufs[j], sg[j])

    def g_wait(c, j):
      pltpu.make_async_copy(y_hbm.at[sidx(c)], bufs[j], sg[j]).wait()

    def s_start(c, j):
      pltpu.async_copy(bufs[j], acc_sh.at[didx(c)], ss[j], add=True)

    def s_wait(c, j):
      pltpu.make_async_copy(bufs[j], acc_sh.at[didx(c)], ss[j]).wait()

    for h in range(halves):
      if h > 0:
        e0 = base + h * hept
        pltpu.sync_copy(src_hbm.at[pl.ds(e0, hept)], src_v)
        pltpu.sync_copy(dst_hbm.at[pl.ds(e0, hept)], dst_v)
      if feature_split:
        off = core * N

        def addoff(i, _):
          src_v[pl.ds(i * 16, 16)] = src_v[pl.ds(i * 16, 16)] + off
          return 0
        lax.fori_loop(0, hept // 16, addoff, 0)
      if h == 0:
        plsc.subcore_barrier()

      g_start(0, 0)
      g_start(1, 1)

      def pipe(i, _):
        for j in range(3):
          c = 3 * i + j
          g_wait(c, j)

          @pl.when(c > 0)
          def _():
            s_wait(c - 1, (j + 2) % 3)

          @pl.when(c + 2 < nchk)
          def _():
            g_start(c + 2, (j + 2) % 3)
          s_start(c, j)
        return 0
      lax.fori_loop(0, npip, pipe, 0)
      for c in range(3 * npip, nchk):
        j = c % 3
        g_wait(c, j)
        s_wait(c - 1, (j + 2) % 3)
        s_start(c, j)
      s_wait(nchk - 1, (nchk - 1) % 3)
    plsc.subcore_barrier()
    _row_chunks(tid, lambda r0: pltpu.async_copy(
        acc_sh.at[pl.ds(r0, CH), :], out_hbm.at[core, pl.ds(r0, CH), :], sw))
    _row_chunks(tid, lambda r0: pltpu.make_async_copy(
        acc_sh.at[pl.ds(r0, CH), :], out_hbm.at[core, pl.ds(r0, CH), :],
        sw).wait())

  return k(y, src, dst)


BLK = 400


def _tc_layer1(x, w1, degp):
  def body(x_ref, w_ref, dp_ref, y_ref, ds_ref):
    deg = dp_ref[0, :, 0:1] + dp_ref[1, :, 0:1] + 1.0
    ds = lax.rsqrt(deg)
    xw = jnp.dot(x_ref[...], w_ref[...], preferred_element_type=jnp.float32)
    y = xw * ds
    y_ref[0, :, :] = y[:, :128]
    y_ref[1, :, :] = y[:, 128:]
    ds_ref[...] = ds

  return pl.pallas_call(
      body,
      grid=(N // BLK,),
      in_specs=[
          pl.BlockSpec((BLK, IN_D), lambda i: (i, 0)),
          pl.BlockSpec((IN_D, HID), lambda i: (0, 0)),
          pl.BlockSpec((NC, BLK, 128), lambda i: (0, i, 0)),
      ],
      out_specs=[
          pl.BlockSpec((NC, BLK, 128), lambda i: (0, i, 0)),
          pl.BlockSpec((BLK, 1), lambda i: (i, 0)),
      ],
      out_shape=[
          jax.ShapeDtypeStruct((NC, N, 128), jnp.float32),
          jax.ShapeDtypeStruct((N, 1), jnp.float32),
      ],
  )(x, w1, degp)


def _tc_layer2(ds, y1pair, acc1, b1r, w2):
  def body(ds_ref, y1_ref, a1_ref, b1_ref, w_ref, y2_ref):
    ds = ds_ref[...]
    pre = (a1_ref[...] + y1_ref[...]) * ds[None, :, :] + b1_ref[...]
    h = jnp.maximum(pre, 0.0)
    hf = jnp.concatenate([h[0], h[1]], axis=1)
    z = jnp.dot(hf, w_ref[...], preferred_element_type=jnp.float32)
    y2_ref[...] = z * ds

  return pl.pallas_call(
      body,
      grid=(N // BLK,),
      in_specs=[
          pl.BlockSpec((BLK, 1), lambda i: (i, 0)),
          pl.BlockSpec((NC, BLK, 128), lambda i: (0, i, 0)),
          pl.BlockSpec((NC, BLK, 128), lambda i: (0, i, 0)),
          pl.BlockSpec((NC, 1, 128), lambda i: (0, 0, 0)),
          pl.BlockSpec((HID, OUT_D), lambda i: (0, 0)),
      ],
      out_specs=pl.BlockSpec((BLK, OUT_D), lambda i: (i, 0)),
      out_shape=jax.ShapeDtypeStruct((N, OUT_D), jnp.float32),
  )(ds, y1pair, acc1, b1r, w2)


def _tc_layer3(ds, acc2, y2, b2r):
  def body(ds_ref, a2_ref, y2_ref, b2_ref, o_ref):
    o_ref[...] = ((a2_ref[0] + a2_ref[1] + y2_ref[...]) * ds_ref[...]
                  + b2_ref[...])

  return pl.pallas_call(
      body,
      grid=(N // BLK,),
      in_specs=[
          pl.BlockSpec((BLK, 1), lambda i: (i, 0)),
          pl.BlockSpec((NC, BLK, OUT_D), lambda i: (0, i, 0)),
          pl.BlockSpec((BLK, OUT_D), lambda i: (i, 0)),
          pl.BlockSpec((1, OUT_D), lambda i: (0, 0)),
      ],
      out_specs=pl.BlockSpec((BLK, OUT_D), lambda i: (i, 0)),
      out_shape=jax.ShapeDtypeStruct((N, OUT_D), jnp.float32),
  )(ds, acc2, y2, b2r)


def kernel(x, edge_index, W1, b1, W2, b2):
  src = edge_index[0].astype(jnp.int32)
  dst = edge_index[1].astype(jnp.int32)
  degp = _deg_partials(dst)
  y1pair, ds = _tc_layer1(x, W1, degp)
  acc1 = _agg(y1pair.reshape(NC * N, 128), src, dst, feature_split=True)
  y2 = _tc_layer2(ds, y1pair, acc1, b1.reshape(NC, 1, 128), W2)
  acc2 = _agg(y2, src, dst, feature_split=False)
  return _tc_layer3(ds, acc2, y2, b2.reshape(1, OUT_D))


# speedup vs baseline: 28.0120x; 1.0728x over previous
import functools

import jax
import jax.numpy as jnp
from jax import lax
from jax.experimental import pallas as pl
from jax.experimental.pallas import tpu as pltpu
from jax.experimental.pallas import tpu_sc as plsc

N = 10000
E = 320000
IN_D = 128
HID = 256
OUT_D = 128

NC = 2
NS = 16
CH = 80
NRC = N // CH


def _mesh():
  return plsc.VectorSubcoreMesh(core_axis_name="c", subcore_axis_name="s",
                                num_cores=NC, num_subcores=NS)


def _row_chunks(tid, fn):
  for j in range(-(-NRC // NS)):
    idx = j * NS + tid

    @pl.when(idx < NRC)
    def _():
      fn(idx * CH)


def _fill(ref, n_rows, width, value):
  def body(i, _):
    for j in range(width // 16):
      ref[i, pl.ds(j * 16, 16)] = jnp.full((16,), value, jnp.float32)
    return 0
  lax.fori_loop(0, n_rows, body, 0)


def _deg_partials(dst):
  ept = E // (NC * NS)
  nchk = ept // CH
  grp = 8

  @functools.partial(
      pl.kernel,
      out_type=jax.ShapeDtypeStruct((NC, N, 128), jnp.float32),
      mesh=_mesh(),
      scratch_types=[
          pltpu.VMEM((ept,), jnp.int32),
          pltpu.VMEM((CH, 128), jnp.float32),
          pltpu.VMEM((CH, 128), jnp.float32),
          pltpu.VMEM_SHARED((N, 128), jnp.float32),
          pltpu.SemaphoreType.DMA,
          pltpu.SemaphoreType.DMA,
      ],
  )
  def k(dst_hbm, out_hbm, idx_v, ones_v, zero_v, acc_sh, sem, sidxl):
    core = lax.axis_index("c")
    tid = lax.axis_index("s")

    def didx(c):
      return idx_v.at[pl.ds(c * CH, CH)]

    idx_src = dst_hbm.at[pl.ds((core * NS + tid) * ept, ept)]
    pltpu.async_copy(idx_src, idx_v, sidxl)
    _fill(ones_v, CH, 128, 1.0)
    _fill(zero_v, CH, 128, 0.0)
    _row_chunks(tid, lambda r0: pltpu.async_copy(
        zero_v, acc_sh.at[pl.ds(r0, CH), :], sem))
    _row_chunks(tid, lambda r0: pltpu.make_async_copy(
        zero_v, acc_sh.at[pl.ds(r0, CH), :], sem).wait())
    pltpu.make_async_copy(idx_src, idx_v, sidxl).wait()
    plsc.subcore_barrier()

    def group(g, _):
      for j in range(grp):
        pltpu.async_copy(ones_v, acc_sh.at[didx(g * grp + j)], sem, add=True)
      for j in range(grp):
        pltpu.make_async_copy(ones_v, acc_sh.at[didx(g * grp + j)],
                              sem).wait()
      return 0
    lax.fori_loop(0, nchk // grp, group, 0)
    for c in range(nchk - nchk % grp, nchk):
      pltpu.async_copy(ones_v, acc_sh.at[didx(c)], sem, add=True)
    for c in range(nchk - nchk % grp, nchk):
      pltpu.make_async_copy(ones_v, acc_sh.at[didx(c)], sem).wait()
    plsc.subcore_barrier()
    _row_chunks(tid, lambda r0: pltpu.async_copy(
        acc_sh.at[pl.ds(r0, CH), :], out_hbm.at[core, pl.ds(r0, CH), :],
        sem))
    _row_chunks(tid, lambda r0: pltpu.make_async_copy(
        acc_sh.at[pl.ds(r0, CH), :], out_hbm.at[core, pl.ds(r0, CH), :],
        sem).wait())

  return k(dst)


def _agg(y, src, dst, feature_split):
  ept = E // NS if feature_split else E // (NC * NS)
  halves = 2 if feature_split else 1
  hept = ept // halves
  nchk = hept // CH
  npip = (nchk - 2) // 3

  @functools.partial(
      pl.kernel,
      out_type=jax.ShapeDtypeStruct((NC, N, 128), jnp.float32),
      mesh=_mesh(),
      scratch_types=[
          pltpu.VMEM((hept,), jnp.int32),
          pltpu.VMEM((hept,), jnp.int32),
          pltpu.VMEM((CH, 128), jnp.float32),
          pltpu.VMEM((CH, 128), jnp.float32),
          pltpu.VMEM((CH, 128), jnp.float32),
          pltpu.VMEM_SHARED((N, 128), jnp.float32),
          [pltpu.SemaphoreType.DMA] * 3,
          [pltpu.SemaphoreType.DMA] * 3,
          pltpu.SemaphoreType.DMA,
      ],
  )
  def k(y_hbm, src_hbm, dst_hbm, out_hbm, src_v, dst_v, b0, b1, b2,
        acc_sh, sg, ss, sw):
    core = lax.axis_index("c")
    tid = lax.axis_index("s")
    bufs = (b0, b1, b2)

    def sidx(c):
      return src_v.at[pl.ds(c * CH, CH)]

    def didx(c):
      return dst_v.at[pl.ds(c * CH, CH)]

    base = tid * ept if feature_split else (core * NS + tid) * ept
    pltpu.async_copy(src_hbm.at[pl.ds(base, hept)], src_v, sg[0])
    pltpu.async_copy(dst_hbm.at[pl.ds(base, hept)], dst_v, sg[1])
    _fill(b0, CH, 128, 0.0)
    _row_chunks(tid, lambda r0: pltpu.async_copy(
        b0, acc_sh.at[pl.ds(r0, CH), :], sw))
    _row_chunks(tid, lambda r0: pltpu.make_async_copy(
        b0, acc_sh.at[pl.ds(r0, CH), :], sw).wait())
    pltpu.make_async_copy(src_hbm.at[pl.ds(base, hept)], src_v, sg[0]).wait()
    pltpu.make_async_copy(dst_hbm.at[pl.ds(base, hept)], dst_v, sg[1]).wait()

    def g_start(c, j):
      pltpu.async_copy(y_hbm.at[sidx(c)], bufs[j], sg[j])

    def g_wait(c, j):
      pltpu.make_async_copy(y_hbm.at[sidx(c)], bufs[j], sg[j]).wait()

    def s_start(c, j):
      pltpu.async_copy(bufs[j], acc_sh.at[didx(c)], ss[j], add=True)

    def s_wait(c, j):
      pltpu.make_async_copy(bufs[j], acc_sh.at[didx(c)], ss[j]).wait()

    for h in range(halves):
      if h > 0:
        e0 = base + h * hept
        pltpu.sync_copy(src_hbm.at[pl.ds(e0, hept)], src_v)
        pltpu.sync_copy(dst_hbm.at[pl.ds(e0, hept)], dst_v)
      if feature_split:
        off = core * N

        def addoff(i, _):
          src_v[pl.ds(i * 16, 16)] = src_v[pl.ds(i * 16, 16)] + off
          return 0
        lax.fori_loop(0, hept // 16, addoff, 0)
      if h == 0:
        plsc.subcore_barrier()

      g_start(0, 0)
      g_start(1, 1)

      def pipe(i, _):
        for j in range(3):
          c = 3 * i + j
          g_wait(c, j)

          @pl.when(c > 0)
          def _():
            s_wait(c - 1, (j + 2) % 3)

          @pl.when(c + 2 < nchk)
          def _():
            g_start(c + 2, (j + 2) % 3)
          s_start(c, j)
        return 0
      lax.fori_loop(0, npip, pipe, 0)
      for c in range(3 * npip, nchk):
        j = c % 3
        g_wait(c, j)
        s_wait(c - 1, (j + 2) % 3)
        s_start(c, j)
      s_wait(nchk - 1, (nchk - 1) % 3)
    plsc.subcore_barrier()
    _row_chunks(tid, lambda r0: pltpu.async_copy(
        acc_sh.at[pl.ds(r0, CH), :], out_hbm.at[core, pl.ds(r0, CH), :], sw))
    _row_chunks(tid, lambda r0: pltpu.make_async_copy(
        acc_sh.at[pl.ds(r0, CH), :], out_hbm.at[core, pl.ds(r0, CH), :],
        sw).wait())

  return k(y, src, dst)


BLK = 2000


def _tc_layer1(x, w1, degp):
  def body(x_ref, w_ref, dp_ref, y_ref, ds_ref):
    deg = dp_ref[0, :, 0:1] + dp_ref[1, :, 0:1] + 1.0
    ds = lax.rsqrt(deg)
    xw = jnp.dot(x_ref[...], w_ref[...], preferred_element_type=jnp.float32)
    y = xw * ds
    y_ref[0, :, :] = y[:, :128]
    y_ref[1, :, :] = y[:, 128:]
    ds_ref[...] = ds

  return pl.pallas_call(
      body,
      grid=(N // BLK,),
      in_specs=[
          pl.BlockSpec((BLK, IN_D), lambda i: (i, 0)),
          pl.BlockSpec((IN_D, HID), lambda i: (0, 0)),
          pl.BlockSpec((NC, BLK, 128), lambda i: (0, i, 0)),
      ],
      out_specs=[
          pl.BlockSpec((NC, BLK, 128), lambda i: (0, i, 0)),
          pl.BlockSpec((BLK, 1), lambda i: (i, 0)),
      ],
      out_shape=[
          jax.ShapeDtypeStruct((NC, N, 128), jnp.float32),
          jax.ShapeDtypeStruct((N, 1), jnp.float32),
      ],
  )(x, w1, degp)


def _tc_layer2(ds, y1pair, acc1, b1r, w2):
  def body(ds_ref, y1_ref, a1_ref, b1_ref, w_ref, y2_ref):
    ds = ds_ref[...]
    pre = (a1_ref[...] + y1_ref[...]) * ds[None, :, :] + b1_ref[...]
    h = jnp.maximum(pre, 0.0)
    hf = jnp.concatenate([h[0], h[1]], axis=1)
    z = jnp.dot(hf, w_ref[...], preferred_element_type=jnp.float32)
    y2_ref[...] = z * ds

  return pl.pallas_call(
      body,
      grid=(N // BLK,),
      in_specs=[
          pl.BlockSpec((BLK, 1), lambda i: (i, 0)),
          pl.BlockSpec((NC, BLK, 128), lambda i: (0, i, 0)),
          pl.BlockSpec((NC, BLK, 128), lambda i: (0, i, 0)),
          pl.BlockSpec((NC, 1, 128), lambda i: (0, 0, 0)),
          pl.BlockSpec((HID, OUT_D), lambda i: (0, 0)),
      ],
      out_specs=pl.BlockSpec((BLK, OUT_D), lambda i: (i, 0)),
      out_shape=jax.ShapeDtypeStruct((N, OUT_D), jnp.float32),
  )(ds, y1pair, acc1, b1r, w2)


def _tc_layer3(ds, acc2, y2, b2r):
  def body(ds_ref, a2_ref, y2_ref, b2_ref, o_ref):
    o_ref[...] = ((a2_ref[0] + a2_ref[1] + y2_ref[...]) * ds_ref[...]
                  + b2_ref[...])

  return pl.pallas_call(
      body,
      grid=(N // BLK,),
      in_specs=[
          pl.BlockSpec((BLK, 1), lambda i: (i, 0)),
          pl.BlockSpec((NC, BLK, OUT_D), lambda i: (0, i, 0)),
          pl.BlockSpec((BLK, OUT_D), lambda i: (i, 0)),
          pl.BlockSpec((1, OUT_D), lambda i: (0, 0)),
      ],
      out_specs=pl.BlockSpec((BLK, OUT_D), lambda i: (i, 0)),
      out_shape=jax.ShapeDtypeStruct((N, OUT_D), jnp.float32),
  )(ds, acc2, y2, b2r)


def kernel(x, edge_index, W1, b1, W2, b2):
  src = edge_index[0].astype(jnp.int32)
  dst = edge_index[1].astype(jnp.int32)
  degp = _deg_partials(dst)
  y1pair, ds = _tc_layer1(x, W1, degp)
  acc1 = _agg(y1pair.reshape(NC * N, 128), src, dst, feature_split=True)
  y2 = _tc_layer2(ds, y1pair, acc1, b1.reshape(NC, 1, 128), W2)
  acc2 = _agg(y2, src, dst, feature_split=False)
  return _tc_layer3(ds, acc2, y2, b2.reshape(1, OUT_D))


# speedup vs baseline: 28.1155x; 1.0037x over previous
import functools

import jax
import jax.numpy as jnp
from jax import lax
from jax.experimental import pallas as pl
from jax.experimental.pallas import tpu as pltpu
from jax.experimental.pallas import tpu_sc as plsc

N = 10000
E = 320000
IN_D = 128
HID = 256
OUT_D = 128

NC = 2
NS = 16
CH = 80
NRC = N // CH


def _mesh():
  return plsc.VectorSubcoreMesh(core_axis_name="c", subcore_axis_name="s",
                                num_cores=NC, num_subcores=NS)


def _row_chunks(tid, fn):
  for j in range(-(-NRC // NS)):
    idx = j * NS + tid

    @pl.when(idx < NRC)
    def _():
      fn(idx * CH)


def _fill(ref, n_rows, width, value):
  def body(i, _):
    for j in range(width // 16):
      ref[i, pl.ds(j * 16, 16)] = jnp.full((16,), value, jnp.float32)
    return 0
  lax.fori_loop(0, n_rows, body, 0)


def _deg_partials(dst):
  ept = E // (NC * NS)
  nchk = ept // CH
  grp = 8

  @functools.partial(
      pl.kernel,
      out_type=jax.ShapeDtypeStruct((NC, N, 128), jnp.float32),
      mesh=_mesh(),
      scratch_types=[
          pltpu.VMEM((ept,), jnp.int32),
          pltpu.VMEM((CH, 128), jnp.float32),
          pltpu.VMEM((CH, 128), jnp.float32),
          pltpu.VMEM_SHARED((N, 128), jnp.float32),
          pltpu.SemaphoreType.DMA,
          pltpu.SemaphoreType.DMA,
      ],
  )
  def k(dst_hbm, out_hbm, idx_v, ones_v, zero_v, acc_sh, sem, sidxl):
    core = lax.axis_index("c")
    tid = lax.axis_index("s")

    def didx(c):
      return idx_v.at[pl.ds(c * CH, CH)]

    idx_src = dst_hbm.at[pl.ds((core * NS + tid) * ept, ept)]
    pltpu.async_copy(idx_src, idx_v, sidxl)
    _fill(ones_v, CH, 128, 1.0)
    _fill(zero_v, CH, 128, 0.0)
    _row_chunks(tid, lambda r0: pltpu.async_copy(
        zero_v, acc_sh.at[pl.ds(r0, CH), :], sem))
    _row_chunks(tid, lambda r0: pltpu.make_async_copy(
        zero_v, acc_sh.at[pl.ds(r0, CH), :], sem).wait())
    pltpu.make_async_copy(idx_src, idx_v, sidxl).wait()
    plsc.subcore_barrier()

    def group(g, _):
      for j in range(grp):
        pltpu.async_copy(ones_v, acc_sh.at[didx(g * grp + j)], sem, add=True)
      for j in range(grp):
        pltpu.make_async_copy(ones_v, acc_sh.at[didx(g * grp + j)],
                              sem).wait()
      return 0
    lax.fori_loop(0, nchk // grp, group, 0)
    for c in range(nchk - nchk % grp, nchk):
      pltpu.async_copy(ones_v, acc_sh.at[didx(c)], sem, add=True)
    for c in range(nchk - nchk % grp, nchk):
      pltpu.make_async_copy(ones_v, acc_sh.at[didx(c)], sem).wait()
    plsc.subcore_barrier()
    _row_chunks(tid, lambda r0: pltpu.async_copy(
        acc_sh.at[pl.ds(r0, CH), :], out_hbm.at[core, pl.ds(r0, CH), :],
        sem))
    _row_chunks(tid, lambda r0: pltpu.make_async_copy(
        acc_sh.at[pl.ds(r0, CH), :], out_hbm.at[core, pl.ds(r0, CH), :],
        sem).wait())

  return k(dst)


def _agg(y, src, dst, feature_split):
  ept = E // NS if feature_split else E // (NC * NS)
  halves = 2 if feature_split else 1
  hept = ept // halves
  nchk = hept // CH
  npip = (nchk - 2) // 3

  @functools.partial(
      pl.kernel,
      out_type=jax.ShapeDtypeStruct((NC, N, 128), jnp.float32),
      mesh=_mesh(),
      scratch_types=[
          pltpu.VMEM((hept,), jnp.int32),
          pltpu.VMEM((hept,), jnp.int32),
          pltpu.VMEM((CH, 128), jnp.float32),
          pltpu.VMEM((CH, 128), jnp.float32),
          pltpu.VMEM((CH, 128), jnp.float32),
          pltpu.VMEM_SHARED((N, 128), jnp.float32),
          [pltpu.SemaphoreType.DMA] * 3,
          [pltpu.SemaphoreType.DMA] * 3,
          pltpu.SemaphoreType.DMA,
      ],
  )
  def k(y_hbm, src_hbm, dst_hbm, out_hbm, src_v, dst_v, b0, b1, b2,
        acc_sh, sg, ss, sw):
    core = lax.axis_index("c")
    tid = lax.axis_index("s")
    bufs = (b0, b1, b2)

    def sidx(c):
      return src_v.at[pl.ds(c * CH, CH)]

    def didx(c):
      return dst_v.at[pl.ds(c * CH, CH)]

    base = tid * ept if feature_split else (core * NS + tid) * ept
    pltpu.async_copy(src_hbm.at[pl.ds(base, hept)], src_v, sg[0])
    pltpu.async_copy(dst_hbm.at[pl.ds(base, hept)], dst_v, sg[1])
    _fill(b0, CH, 128, 0.0)
    _row_chunks(tid, lambda r0: pltpu.async_copy(
        b0, acc_sh.at[pl.ds(r0, CH), :], sw))
    _row_chunks(tid, lambda r0: pltpu.make_async_copy(
        b0, acc_sh.at[pl.ds(r0, CH), :], sw).wait())
    pltpu.make_async_copy(src_hbm.at[pl.ds(base, hept)], src_v, sg[0]).wait()
    pltpu.make_async_copy(dst_hbm.at[pl.ds(base, hept)], dst_v, sg[1]).wait()

    def g_start(c, j):
      pltpu.async_copy(y_hbm.at[sidx(c)], bufs[j], sg[j])

    def g_wait(c, j):
      pltpu.make_async_copy(y_hbm.at[sidx(c)], bufs[j], sg[j]).wait()

    def s_start(c, j):
      pltpu.async_copy(bufs[j], acc_sh.at[didx(c)], ss[j], add=True)

    def s_wait(c, j):
      pltpu.make_async_copy(bufs[j], acc_sh.at[didx(c)], ss[j]).wait()

    for h in range(halves):
      if h > 0:
        e0 = base + h * hept
        pltpu.sync_copy(src_hbm.at[pl.ds(e0, hept)], src_v)
        pltpu.sync_copy(dst_hbm.at[pl.ds(e0, hept)], dst_v)
      if feature_split:
        off = core * N

        def addoff(i, _):
          src_v[pl.ds(i * 16, 16)] = src_v[pl.ds(i * 16, 16)] + off
          return 0
        lax.fori_loop(0, hept // 16, addoff, 0)
      if h == 0:
        plsc.subcore_barrier()

      g_start(0, 0)
      g_start(1, 1)

      def pipe(i, _):
        for j in range(3):
          c = 3 * i + j
          g_wait(c, j)

          @pl.when(c > 0)
          def _():
            s_wait(c - 1, (j + 2) % 3)

          @pl.when(c + 2 < nchk)
          def _():
            g_start(c + 2, (j + 2) % 3)
          s_start(c, j)
        return 0
      lax.fori_loop(0, npip, pipe, 0)
      for c in range(3 * npip, nchk):
        j = c % 3
        g_wait(c, j)
        s_wait(c - 1, (j + 2) % 3)
        s_start(c, j)
      s_wait(nchk - 1, (nchk - 1) % 3)
    plsc.subcore_barrier()
    _row_chunks(tid, lambda r0: pltpu.async_copy(
        acc_sh.at[pl.ds(r0, CH), :], out_hbm.at[core, pl.ds(r0, CH), :], sw))
    _row_chunks(tid, lambda r0: pltpu.make_async_copy(
        acc_sh.at[pl.ds(r0, CH), :], out_hbm.at[core, pl.ds(r0, CH), :],
        sw).wait())

  return k(y, src, dst)


BLK = 5000


def _tc_layer1(x, w1, degp):
  def body(x_ref, w_ref, dp_ref, y_ref, ds_ref):
    deg = dp_ref[0, :, 0:1] + dp_ref[1, :, 0:1] + 1.0
    ds = lax.rsqrt(deg)
    xw = jnp.dot(x_ref[...], w_ref[...], preferred_element_type=jnp.float32)
    y = xw * ds
    y_ref[0, :, :] = y[:, :128]
    y_ref[1, :, :] = y[:, 128:]
    ds_ref[...] = ds

  return pl.pallas_call(
      body,
      grid=(N // BLK,),
      in_specs=[
          pl.BlockSpec((BLK, IN_D), lambda i: (i, 0)),
          pl.BlockSpec((IN_D, HID), lambda i: (0, 0)),
          pl.BlockSpec((NC, BLK, 128), lambda i: (0, i, 0)),
      ],
      out_specs=[
          pl.BlockSpec((NC, BLK, 128), lambda i: (0, i, 0)),
          pl.BlockSpec((BLK, 1), lambda i: (i, 0)),
      ],
      out_shape=[
          jax.ShapeDtypeStruct((NC, N, 128), jnp.float32),
          jax.ShapeDtypeStruct((N, 1), jnp.float32),
      ],
  )(x, w1, degp)


def _tc_layer2(ds, y1pair, acc1, b1r, w2):
  def body(ds_ref, y1_ref, a1_ref, b1_ref, w_ref, y2_ref):
    ds = ds_ref[...]
    pre = (a1_ref[...] + y1_ref[...]) * ds[None, :, :] + b1_ref[...]
    h = jnp.maximum(pre, 0.0)
    hf = jnp.concatenate([h[0], h[1]], axis=1)
    z = jnp.dot(hf, w_ref[...], preferred_element_type=jnp.float32)
    y2_ref[...] = z * ds

  return pl.pallas_call(
      body,
      grid=(N // BLK,),
      in_specs=[
          pl.BlockSpec((BLK, 1), lambda i: (i, 0)),
          pl.BlockSpec((NC, BLK, 128), lambda i: (0, i, 0)),
          pl.BlockSpec((NC, BLK, 128), lambda i: (0, i, 0)),
          pl.BlockSpec((NC, 1, 128), lambda i: (0, 0, 0)),
          pl.BlockSpec((HID, OUT_D), lambda i: (0, 0)),
      ],
      out_specs=pl.BlockSpec((BLK, OUT_D), lambda i: (i, 0)),
      out_shape=jax.ShapeDtypeStruct((N, OUT_D), jnp.float32),
  )(ds, y1pair, acc1, b1r, w2)


def _tc_layer3(ds, acc2, y2, b2r):
  def body(ds_ref, a2_ref, y2_ref, b2_ref, o_ref):
    o_ref[...] = ((a2_ref[0] + a2_ref[1] + y2_ref[...]) * ds_ref[...]
                  + b2_ref[...])

  return pl.pallas_call(
      body,
      grid=(N // BLK,),
      in_specs=[
          pl.BlockSpec((BLK, 1), lambda i: (i, 0)),
          pl.BlockSpec((NC, BLK, OUT_D), lambda i: (0, i, 0)),
          pl.BlockSpec((BLK, OUT_D), lambda i: (i, 0)),
          pl.BlockSpec((1, OUT_D), lambda i: (0, 0)),
      ],
      out_specs=pl.BlockSpec((BLK, OUT_D), lambda i: (i, 0)),
      out_shape=jax.ShapeDtypeStruct((N, OUT_D), jnp.float32),
  )(ds, acc2, y2, b2r)


def kernel(x, edge_index, W1, b1, W2, b2):
  src = edge_index[0].astype(jnp.int32)
  dst = edge_index[1].astype(jnp.int32)
  degp = _deg_partials(dst)
  y1pair, ds = _tc_layer1(x, W1, degp)
  acc1 = _agg(y1pair.reshape(NC * N, 128), src, dst, feature_split=True)
  y2 = _tc_layer2(ds, y1pair, acc1, b1.reshape(NC, 1, 128), W2)
  acc2 = _agg(y2, src, dst, feature_split=False)
  return _tc_layer3(ds, acc2, y2, b2.reshape(1, OUT_D))
